# baseline (speedup 1.0000x reference)
import functools

import jax
import jax.numpy as jnp
from jax import lax
from jax.experimental import pallas as pl
from jax.experimental.pallas import tpu as pltpu
from jax.experimental.pallas import tpu_sc as plsc

CV = 64
BO = 512


def _sc_pool_body(nv_hbm, x_hbm, o_hbm, nv_v, buf, stage, sems, *, cv, fb, B, V):
    c = lax.axis_index("c")
    s = lax.axis_index("s")
    wid = s * 2 + c
    f0 = pl.multiple_of(wid * fb, 128)

    pltpu.sync_copy(nv_hbm, nv_v.at[pl.ds(0, nv_hbm.shape[0])])

    def do_sample(b, carry):
        nv = jnp.minimum(nv_v[pl.ds(b, 16)][0], V)
        nchunks = (nv + cv - 1) // cv
        last0 = jnp.maximum(0, ((nv - cv + 7) // 8) * 8)

        def row0(i):
            return pl.multiple_of(jnp.minimum(i * cv, last0), 8)

        def start(i, slot):
            pltpu.make_async_copy(
                x_hbm.at[b, pl.ds(row0(i), cv), pl.ds(f0, fb)],
                buf.at[slot], sems.at[slot]).start()

        start(0, 0)

        def chunk(i, accs):
            slot = lax.rem(i, 2)

            @pl.when(i + 1 < nchunks)
            def _prefetch():
                start(i + 1, 1 - slot)

            pltpu.make_async_copy(
                x_hbm.at[b, pl.ds(0, cv), pl.ds(f0, fb)],
                buf.at[slot], sems.at[slot]).wait()
            nrows = jnp.minimum(cv, nv - row0(i))

            def rowstep(r, accs):
                return tuple(
                    jnp.maximum(a, buf[slot, r, pl.ds(f * 16, 16)])
                    for f, a in enumerate(accs)
                )

            return lax.fori_loop(0, nrows, rowstep, accs)

        neg = jnp.full((16,), -jnp.inf, jnp.float32)
        accs = tuple(neg for _ in range(fb // 16))
        accs = lax.fori_loop(0, nchunks, chunk, accs)
        for f, a in enumerate(accs):
            stage[pl.ds(f * 16, 16)] = a
        pltpu.sync_copy(stage, o_hbm.at[b, pl.ds(f0, fb)])
        return carry

    lax.fori_loop(0, B, do_sample, 0)


def _linear_body(k_ref, w_ref, bias_ref, o_ref):
    out = lax.dot_general(
        k_ref[...], w_ref[...],
        dimension_numbers=(((1,), (1,)), ((), ())),
        preferred_element_type=jnp.float32,
    )
    o_ref[...] = out + bias_ref[...]


def kernel(batch_size, max_num_views, num_views, x, W, b):
    B, V, D = x.shape
    O = W.shape[0]
    info = plsc.get_sparse_core_info()
    nw = info.num_cores * info.num_subcores
    fb = D // nw

    pool = functools.partial(
        pl.kernel,
        mesh=plsc.VectorSubcoreMesh(core_axis_name="c", subcore_axis_name="s"),
        out_type=jax.ShapeDtypeStruct((B, D), jnp.float32),
        scratch_types=[
            pltpu.VMEM((32,), jnp.int32),
            pltpu.VMEM((2, CV, fb), jnp.float32),
            pltpu.VMEM((fb,), jnp.float32),
            pltpu.SemaphoreType.DMA((2,)),
        ],
    )(functools.partial(_sc_pool_body, cv=CV, fb=fb, B=B, V=V))
    k = pool(num_views.astype(jnp.int32), x)

    bias = b.reshape(1, O)
    linear = pl.pallas_call(
        _linear_body,
        grid=(O // BO,),
        in_specs=[
            pl.BlockSpec((B, D), lambda o: (0, 0)),
            pl.BlockSpec((BO, D), lambda o: (o, 0)),
            pl.BlockSpec((1, BO), lambda o: (0, o)),
        ],
        out_specs=pl.BlockSpec((B, BO), lambda o: (0, o)),
        out_shape=jax.ShapeDtypeStruct((B, O), jnp.float32),
        compiler_params=pltpu.CompilerParams(
            dimension_semantics=("arbitrary",),
        ),
    )
    logits = linear(k, W, bias)
    return (logits, k)


# speedup vs baseline: 1.0017x; 1.0017x over previous
import functools

import jax
import jax.numpy as jnp
from jax import lax
from jax.experimental import pallas as pl
from jax.experimental.pallas import tpu as pltpu
from jax.experimental.pallas import tpu_sc as plsc

CV = 64
BO = 512


def _sc_pool_body(nv_hbm, x_hbm, o_hbm, nv_v, buf, stage, sems, *, cv, fb, B, V):
    c = lax.axis_index("c")
    s = lax.axis_index("s")
    wid = s * 2 + c
    f0 = pl.multiple_of(wid * fb, 128)

    pltpu.sync_copy(nv_hbm, nv_v.at[pl.ds(0, nv_hbm.shape[0])])

    def do_sample(b, carry):
        nv = jnp.minimum(nv_v[pl.ds(b, 16)][0], V)
        nchunks = (nv + cv - 1) // cv
        last0 = jnp.maximum(0, ((nv - cv + 7) // 8) * 8)

        def row0(i):
            return pl.multiple_of(jnp.minimum(i * cv, last0), 8)

        def start(i, slot):
            pltpu.make_async_copy(
                x_hbm.at[b, pl.ds(row0(i), cv), pl.ds(f0, fb)],
                buf.at[slot], sems.at[slot]).start()

        start(0, 0)

        def chunk(i, accs):
            slot = lax.rem(i, 2)

            @pl.when(i + 1 < nchunks)
            def _prefetch():
                start(i + 1, 1 - slot)

            pltpu.make_async_copy(
                x_hbm.at[b, pl.ds(0, cv), pl.ds(f0, fb)],
                buf.at[slot], sems.at[slot]).wait()
            nrows = jnp.minimum(cv, nv - row0(i))
            ngroups = nrows // 8

            def group_step(g, accs):
                base = g * 8
                for kk in range(8):
                    accs = tuple(
                        jnp.maximum(a, buf[slot, base + kk, pl.ds(f * 16, 16)])
                        for f, a in enumerate(accs)
                    )
                return accs

            accs = lax.fori_loop(0, ngroups, group_step, accs)

            def rowstep(r, accs):
                return tuple(
                    jnp.maximum(a, buf[slot, r, pl.ds(f * 16, 16)])
                    for f, a in enumerate(accs)
                )

            return lax.fori_loop(ngroups * 8, nrows, rowstep, accs)

        neg = jnp.full((16,), -jnp.inf, jnp.float32)
        accs = tuple(neg for _ in range(fb // 16))
        accs = lax.fori_loop(0, nchunks, chunk, accs)
        for f, a in enumerate(accs):
            stage[pl.ds(f * 16, 16)] = a
        pltpu.sync_copy(stage, o_hbm.at[b, pl.ds(f0, fb)])
        return carry

    lax.fori_loop(0, B, do_sample, 0)


def _linear_body(k_ref, w_ref, bias_ref, o_ref):
    out = lax.dot_general(
        k_ref[...], w_ref[...],
        dimension_numbers=(((1,), (1,)), ((), ())),
        preferred_element_type=jnp.float32,
    )
    o_ref[...] = out + bias_ref[...]


def kernel(batch_size, max_num_views, num_views, x, W, b):
    B, V, D = x.shape
    O = W.shape[0]
    info = plsc.get_sparse_core_info()
    nw = info.num_cores * info.num_subcores
    fb = D // nw

    pool = functools.partial(
        pl.kernel,
        mesh=plsc.VectorSubcoreMesh(core_axis_name="c", subcore_axis_name="s"),
        out_type=jax.ShapeDtypeStruct((B, D), jnp.float32),
        scratch_types=[
            pltpu.VMEM((32,), jnp.int32),
            pltpu.VMEM((2, CV, fb), jnp.float32),
            pltpu.VMEM((fb,), jnp.float32),
            pltpu.SemaphoreType.DMA((2,)),
        ],
    )(functools.partial(_sc_pool_body, cv=CV, fb=fb, B=B, V=V))
    k = pool(num_views.astype(jnp.int32), x)

    bias = b.reshape(1, O)
    linear = pl.pallas_call(
        _linear_body,
        grid=(O // BO,),
        in_specs=[
            pl.BlockSpec((B, D), lambda o: (0, 0)),
            pl.BlockSpec((BO, D), lambda o: (o, 0)),
            pl.BlockSpec((1, BO), lambda o: (0, o)),
        ],
        out_specs=pl.BlockSpec((B, BO), lambda o: (0, o)),
        out_shape=jax.ShapeDtypeStruct((B, O), jnp.float32),
        compiler_params=pltpu.CompilerParams(
            dimension_semantics=("arbitrary",),
        ),
    )
    logits = linear(k, W, bias)
    return (logits, k)
